# baseline (speedup 1.0000x reference)
import functools

import jax
import jax.numpy as jnp
from jax import lax
from jax.experimental import pallas as pl
from jax.experimental.pallas import tpu as pltpu
from jax.experimental.pallas import tpu_sc as plsc

_NUM_CATEGORIES = 1000
_USER_DIM = 64
_MOVIE_DIM = 32
_CAT_DIM = 32
_BATCH = 16384
_HIST = 20
_MARGIN = 0.1

_NC = 2
_NS = 16
_NW = _NC * _NS
_BPW = _BATCH // _NW
_CHUNK = 128
_NCHUNK = _BPW // _CHUNK
_L = 16
_NBLK = _BPW // _L


def _body(uid_hbm, mid_hbm, cats_hbm, eu_hbm, bu_hbm, em_hbm, ec_hbm,
          bm_hbm, out_hbm,
          uid_v, mid_v, cats_v, urows_v, mrows_v, ub_v, mb_v, ctab_v,
          out_v, sem):
    wid = lax.axis_index("s") * _NC + lax.axis_index("c")

    pltpu.sync_copy(uid_hbm.at[pl.ds(wid * _NCHUNK, _NCHUNK)], uid_v)
    pltpu.sync_copy(mid_hbm.at[pl.ds(wid * _NCHUNK, _NCHUNK)], mid_v)
    pltpu.sync_copy(cats_hbm.at[pl.ds(wid * _BPW * _HIST, _BPW * _HIST)],
                    cats_v)
    pltpu.sync_copy(ec_hbm, ctab_v)

    copies = []
    for k in range(_NCHUNK):
        dst = pl.ds(k * _CHUNK, _CHUNK)
        copies.append(pltpu.async_copy(eu_hbm.at[uid_v.at[k]],
                                       urows_v.at[dst], sem))
        copies.append(pltpu.async_copy(em_hbm.at[mid_v.at[k]],
                                       mrows_v.at[dst], sem))
        copies.append(pltpu.async_copy(bu_hbm.at[uid_v.at[k]],
                                       ub_v.at[dst], sem))
        copies.append(pltpu.async_copy(bm_hbm.at[mid_v.at[k]],
                                       mb_v.at[dst], sem))
    for c in copies:
        c.wait()

    lanes = lax.iota(jnp.int32, _L)
    zeros_i = jnp.zeros((_L,), jnp.int32)

    def step(t, carry):
        rows = t * _L + lanes
        cat_base = rows * _HIST

        cvecs = [plsc.load_gather(cats_v, [cat_base + j])
                 for j in range(_HIST)]
        cnt = (cvecs[0] != 0).astype(jnp.float32)
        for j in range(1, _HIST):
            cnt = cnt + (cvecs[j] != 0).astype(jnp.float32)
        inv = 1.0 / jnp.maximum(cnt, 1.0)

        acc = (plsc.load_gather(ub_v, [rows, zeros_i])
               + plsc.load_gather(mb_v, [rows, zeros_i]))

        for d in range(_MOVIE_DIM):
            col = jnp.full((_L,), d, jnp.int32)
            u_d = plsc.load_gather(urows_v, [rows, col])
            m_d = plsc.load_gather(mrows_v, [rows, col])
            acc = acc + u_d * m_d

        for d in range(_CAT_DIM):
            col = jnp.full((_L,), d, jnp.int32)
            u_d = plsc.load_gather(urows_v, [rows, jnp.full((_L,), _MOVIE_DIM + d, jnp.int32)])
            s = plsc.load_gather(ctab_v, [cvecs[0], col])
            for j in range(1, _HIST):
                s = s + plsc.load_gather(ctab_v, [cvecs[j], col])
            acc = acc + u_d * (s * inv)

        prob = 1.0 / (1.0 + jnp.exp(-acc))
        out_v[pl.ds(t * _L, _L)] = prob * (1.0 + 2.0 * _MARGIN) - _MARGIN
        return carry

    lax.fori_loop(0, _NBLK, step, 0)
    pltpu.sync_copy(out_v, out_hbm.at[pl.ds(wid * _BPW, _BPW)])


@functools.partial(
    pl.kernel,
    out_type=jax.ShapeDtypeStruct((_BATCH,), jnp.float32),
    mesh=plsc.VectorSubcoreMesh(core_axis_name="c", subcore_axis_name="s",
                                num_cores=_NC, num_subcores=_NS),
    compiler_params=pltpu.CompilerParams(needs_layout_passes=False,
                                         use_tc_tiling_on_sc=False),
    scratch_types=[
        pltpu.VMEM((_NCHUNK, _CHUNK), jnp.int32),
        pltpu.VMEM((_NCHUNK, _CHUNK), jnp.int32),
        pltpu.VMEM((_BPW * _HIST,), jnp.int32),
        pltpu.VMEM((_BPW, _USER_DIM), jnp.float32),
        pltpu.VMEM((_BPW, _MOVIE_DIM), jnp.float32),
        pltpu.VMEM((_BPW, 1), jnp.float32),
        pltpu.VMEM((_BPW, 1), jnp.float32),
        pltpu.VMEM((_NUM_CATEGORIES, _CAT_DIM), jnp.float32),
        pltpu.VMEM((_BPW,), jnp.float32),
        pltpu.SemaphoreType.DMA,
    ],
)
def _sc_kernel(uid_hbm, mid_hbm, cats_hbm, eu_hbm, bu_hbm, em_hbm, ec_hbm,
               bm_hbm, out_hbm,
               uid_v, mid_v, cats_v, urows_v, mrows_v, ub_v, mb_v, ctab_v,
               out_v, sem):
    _body(uid_hbm, mid_hbm, cats_hbm, eu_hbm, bu_hbm, em_hbm, ec_hbm,
          bm_hbm, out_hbm,
          uid_v, mid_v, cats_v, urows_v, mrows_v, ub_v, mb_v, ctab_v,
          out_v, sem)


def kernel(user_id, movie_id, movie_categories, emb_users, bias_user,
           emb_movies, emb_movie_cats, bias_movie):
    uid2 = user_id.reshape(_BATCH // _CHUNK, _CHUNK)
    mid2 = movie_id.reshape(_BATCH // _CHUNK, _CHUNK)
    cats_flat = movie_categories.reshape(-1)
    return _sc_kernel(uid2, mid2, cats_flat, emb_users, bias_user,
                      emb_movies, emb_movie_cats, bias_movie)


# speedup vs baseline: 2.7040x; 2.7040x over previous
import functools

import jax
import jax.numpy as jnp
from jax import lax
from jax.experimental import pallas as pl
from jax.experimental.pallas import tpu as pltpu
from jax.experimental.pallas import tpu_sc as plsc

_NUM_CATEGORIES = 1000
_USER_DIM = 64
_MOVIE_DIM = 32
_CAT_DIM = 32
_BATCH = 16384
_HIST = 20
_MARGIN = 0.1

_NC = 2
_NS = 16
_NW = _NC * _NS
_BPW = _BATCH // _NW
_CHUNK = 128
_NCHUNK = _BPW // _CHUNK
_L = 16
_NBLK = _BPW // _L


@functools.partial(
    pl.kernel,
    out_type=jax.ShapeDtypeStruct((_BATCH,), jnp.float32),
    mesh=plsc.VectorSubcoreMesh(core_axis_name="c", subcore_axis_name="s",
                                num_cores=_NC, num_subcores=_NS),
    compiler_params=pltpu.CompilerParams(needs_layout_passes=False,
                                         use_tc_tiling_on_sc=False),
    scratch_types=[
        pltpu.VMEM((_NCHUNK, _CHUNK), jnp.int32),
        pltpu.VMEM((_NCHUNK, _CHUNK), jnp.int32),
        pltpu.VMEM((_BPW * _HIST,), jnp.int32),
        pltpu.VMEM((_BPW, _USER_DIM), jnp.float32),
        pltpu.VMEM((_BPW, _MOVIE_DIM), jnp.float32),
        pltpu.VMEM((_BPW,), jnp.float32),
        pltpu.VMEM((_BPW,), jnp.float32),
        pltpu.VMEM((_NUM_CATEGORIES, _CAT_DIM), jnp.float32),
        pltpu.VMEM((_BPW,), jnp.float32),
        pltpu.SemaphoreType.DMA,
    ],
)
def _sc_kernel(uid_hbm, mid_hbm, cats_hbm, eu_hbm, bu_hbm, em_hbm, ec_hbm,
               bm_hbm, out_hbm,
               uid_v, mid_v, cats_v, urows_v, mrows_v, ub_v, mb_v, ctab_v,
               out_v, sem):
    wid = lax.axis_index("s") * _NC + lax.axis_index("c")
    base = wid * _BPW

    for k in range(_NCHUNK):
        pltpu.sync_copy(uid_hbm.at[pl.ds(base + k * _CHUNK, _CHUNK)],
                        uid_v.at[k])
        pltpu.sync_copy(mid_hbm.at[pl.ds(base + k * _CHUNK, _CHUNK)],
                        mid_v.at[k])
    pltpu.sync_copy(cats_hbm.at[pl.ds(base * _HIST, _BPW * _HIST)], cats_v)
    pltpu.sync_copy(ec_hbm, ctab_v)

    copies = []
    for k in range(_NCHUNK):
        dst = pl.ds(k * _CHUNK, _CHUNK)
        copies.append(pltpu.async_copy(eu_hbm.at[uid_v.at[k]],
                                       urows_v.at[dst], sem))
        copies.append(pltpu.async_copy(em_hbm.at[mid_v.at[k]],
                                       mrows_v.at[dst], sem))
        copies.append(pltpu.async_copy(bu_hbm.at[uid_v.at[k]],
                                       ub_v.at[dst], sem))
        copies.append(pltpu.async_copy(bm_hbm.at[mid_v.at[k]],
                                       mb_v.at[dst], sem))
    for c in copies:
        c.wait()

    lanes = lax.iota(jnp.int32, _L)

    def step(t, carry):
        rows = t * _L + lanes
        cat_base = rows * _HIST

        cvecs = [plsc.load_gather(cats_v, [cat_base + j])
                 for j in range(_HIST)]
        cnt = (cvecs[0] != 0).astype(jnp.float32)
        for j in range(1, _HIST):
            cnt = cnt + (cvecs[j] != 0).astype(jnp.float32)
        inv = 1.0 / jnp.maximum(cnt, 1.0)

        acc = ub_v[pl.ds(t * _L, _L)] + mb_v[pl.ds(t * _L, _L)]

        for dd in range(_CAT_DIM):
            col = (dd + lanes) & (_CAT_DIM - 1)
            u_lo = plsc.load_gather(urows_v, [rows, col])
            m_d = plsc.load_gather(mrows_v, [rows, col])
            acc = acc + u_lo * m_d
            u_hi = plsc.load_gather(urows_v, [rows, col + _MOVIE_DIM])
            s = plsc.load_gather(ctab_v, [cvecs[0], col])
            for j in range(1, _HIST):
                s = s + plsc.load_gather(ctab_v, [cvecs[j], col])
            acc = acc + u_hi * (s * inv)

        prob = 1.0 / (1.0 + jnp.exp(-acc))
        out_v[pl.ds(t * _L, _L)] = prob * (1.0 + 2.0 * _MARGIN) - _MARGIN
        return carry

    lax.fori_loop(0, _NBLK, step, 0)
    pltpu.sync_copy(out_v, out_hbm.at[pl.ds(base, _BPW)])


def kernel(user_id, movie_id, movie_categories, emb_users, bias_user,
           emb_movies, emb_movie_cats, bias_movie):
    cats_flat = movie_categories.reshape(-1)
    bu_flat = bias_user.reshape(-1)
    bm_flat = bias_movie.reshape(-1)
    return _sc_kernel(user_id, movie_id, cats_flat, emb_users, bu_flat,
                      emb_movies, emb_movie_cats, bm_flat)
